# baseline (speedup 1.0000x reference)
import functools

import jax
import jax.numpy as jnp
from jax.experimental import pallas as pl

_F32 = jnp.float32


def _conv_kernel(x_ref, w_ref, b_ref, *out_refs, kh, rows, ow, cout,
                 relu, stats):
    out_ref = out_refs[0]
    x = x_ref[0, 0]
    kwcin = x.shape[2]
    acc = None
    for di in range(kh):
        xs = x[di:di + rows].reshape(rows * ow, kwcin)
        t = jnp.dot(xs, w_ref[di], preferred_element_type=_F32)
        acc = t if acc is None else acc + t
    acc = acc + b_ref[0][None, :]
    if relu:
        acc = jnp.maximum(acc, 0.0)
    out_ref[0, 0] = acc.reshape(rows, ow, cout)
    if stats:
        s_ref, ss_ref = out_refs[1], out_refs[2]
        first = (pl.program_id(0) == 0) & (pl.program_id(1) == 0)

        @pl.when(first)
        def _init():
            s_ref[...] = jnp.zeros(s_ref.shape, _F32)
            ss_ref[...] = jnp.zeros(ss_ref.shape, _F32)

        s_ref[0, :] = s_ref[0, :] + jnp.sum(acc, axis=0)
        ss_ref[0, :] = ss_ref[0, :] + jnp.sum(acc * acc, axis=0)


def _pick_chunk(oh):
    if oh <= 56:
        return oh
    for c in (20, 28, 32, 44, 55, 56):
        if oh % c == 0:
            return c
    return oh


def _pconv(xp, w, b, *, relu=False, stats=False):
    B, Hp, Wp, Cin = xp.shape
    kh, kw, _, Cout = w.shape
    OH, OW = Hp - kh + 1, Wp - kw + 1
    chunk = _pick_chunk(OH)
    nc = OH // chunk
    xw = jnp.concatenate(
        [jax.lax.slice_in_dim(xp, j, j + OW, axis=2) for j in range(kw)],
        axis=3)
    xc = jnp.stack(
        [jax.lax.slice_in_dim(xw, r, r + chunk + kh - 1, axis=1)
         for r in range(0, OH, chunk)], axis=1)
    b2 = b.reshape(1, Cout)
    wr = w.reshape(kh, kw * Cin, Cout)
    kern = functools.partial(_conv_kernel, kh=kh, rows=chunk, ow=OW,
                             cout=Cout, relu=relu, stats=stats)
    out_shape = [jax.ShapeDtypeStruct((B, nc, chunk, OW, Cout), _F32)]
    out_specs = [pl.BlockSpec((1, 1, chunk, OW, Cout),
                              lambda bi, ri: (bi, ri, 0, 0, 0))]
    if stats:
        for _ in range(2):
            out_shape.append(jax.ShapeDtypeStruct((1, Cout), _F32))
            out_specs.append(pl.BlockSpec((1, Cout), lambda bi, ri: (0, 0)))
    in_specs = [
        pl.BlockSpec((1, 1, chunk + kh - 1, OW, kw * Cin),
                     lambda bi, ri: (bi, ri, 0, 0, 0)),
        pl.BlockSpec(wr.shape, lambda bi, ri: (0, 0, 0)),
        pl.BlockSpec((1, Cout), lambda bi, ri: (0, 0)),
    ]
    res = pl.pallas_call(kern, grid=(B, nc), in_specs=in_specs,
                         out_specs=out_specs, out_shape=out_shape)(xc, wr, b2)
    y = res[0].reshape(B, OH, OW, Cout)
    if stats:
        return y, res[1], res[2]
    return y


def _s2d(x):
    B, H, W, C = x.shape
    xp = jnp.pad(x, ((0, 0), (1, 1), (1, 1), (0, 0)))
    Hs, Ws = (H + 2) // 2, (W + 2) // 2
    return (xp.reshape(B, Hs, 2, Ws, 2, C)
            .transpose(0, 1, 3, 2, 4, 5)
            .reshape(B, Hs, Ws, 4 * C))


def _s2d_weights(w):
    kh, kw, ci, co = w.shape
    return (w.reshape(2, 2, 2, 2, ci, co)
            .transpose(0, 2, 1, 3, 4, 5)
            .reshape(2, 2, 4 * ci, co))


def _enc_conv(x, w, b):
    return _pconv(_s2d(x), _s2d_weights(w), b)


def _vq_kernel(f_ref, c_ref, disc_ref, q_ref, qst_ref):
    x = f_ref[...]
    c = c_ref[...]
    K = c.shape[0]
    xc = jax.lax.dot_general(x, c, (((1,), (1,)), ((), ())),
                             preferred_element_type=_F32)
    d = (jnp.sum(x * x, axis=1, keepdims=True) - 2.0 * xc
         + jnp.sum(c * c, axis=1)[None, :])
    iota = jax.lax.broadcasted_iota(jnp.int32, d.shape, 1)
    dmin = jnp.min(d, axis=1, keepdims=True)
    idx = jnp.min(jnp.where(d == dmin, iota, K), axis=1)
    one_hot = (iota == idx[:, None]).astype(_F32)
    q = jnp.dot(one_hot, c, preferred_element_type=_F32)
    disc_ref[...] = one_hot
    q_ref[...] = q
    qst_ref[...] = x + (q - x)


def _vq(flat, codebook):
    N, D = flat.shape
    K = codebook.shape[0]
    R = N // 8 if (N % 8 == 0 and (N // 8) % 8 == 0) else N
    G = N // R
    outs = [jax.ShapeDtypeStruct((N, K), _F32),
            jax.ShapeDtypeStruct((N, D), _F32),
            jax.ShapeDtypeStruct((N, D), _F32)]
    out_specs = [pl.BlockSpec((R, K), lambda i: (i, 0)),
                 pl.BlockSpec((R, D), lambda i: (i, 0)),
                 pl.BlockSpec((R, D), lambda i: (i, 0))]
    in_specs = [pl.BlockSpec((R, D), lambda i: (i, 0)),
                pl.BlockSpec((K, D), lambda i: (0, 0))]
    return pl.pallas_call(_vq_kernel, grid=(G,), in_specs=in_specs,
                          out_specs=out_specs, out_shape=outs)(flat, codebook)


def _bn_kernel(y_ref, s_ref, ss_ref, g_ref, b_ref, o_ref, *, n):
    m = s_ref[0] / n
    v = ss_ref[0] / n - m * m
    sc = g_ref[0] / jnp.sqrt(v + 1e-5)
    o_ref[...] = jnp.maximum((y_ref[...] - m[None, :]) * sc[None, :]
                             + b_ref[0][None, :], 0.0)


def _bn_relu(y, s, ss, g, bb):
    B, H, W, C = y.shape
    n = float(B * H * W)
    y2 = y.reshape(-1, C)
    N = y2.shape[0]
    G = 16 if (N % 16 == 0 and (N // 16) % 8 == 0) else 1
    R = N // G
    kern = functools.partial(_bn_kernel, n=n)
    vec = pl.BlockSpec((1, C), lambda bi: (0, 0))
    out = pl.pallas_call(
        kern, grid=(G,),
        in_specs=[pl.BlockSpec((R, C), lambda bi: (bi, 0)),
                  vec, vec, vec, vec],
        out_specs=pl.BlockSpec((R, C), lambda bi: (bi, 0)),
        out_shape=jax.ShapeDtypeStruct((N, C), _F32),
    )(y2, s, ss, g.reshape(1, C), bb.reshape(1, C))
    return out.reshape(B, H, W, C)


def _upsample(x):
    return jnp.repeat(jnp.repeat(x, 2, axis=1), 2, axis=2)


def _pad_same4(x):
    return jnp.pad(x, ((0, 0), (1, 2), (1, 2), (0, 0)))


def kernel(img, We1, be1, We2, be2, We3, be3, codebook,
           Wd1, bd1, g1, bb1, Wd2, bd2, g2, bb2, Wd3, bd3, g3, bb3, Wo, bo):
    x = _enc_conv(img, We1, be1)
    x = _enc_conv(x, We2, be2)
    encoded = _enc_conv(x, We3, be3)

    D = codebook.shape[1]
    flat = encoded.reshape(-1, D)
    discrete, quantized_flat, qst_flat = _vq(flat, codebook)
    quantized = quantized_flat.reshape(encoded.shape)
    qst = qst_flat.reshape(encoded.shape)

    y = _upsample(qst)
    y, s, ss = _pconv(_pad_same4(y), Wd1, bd1, stats=True)
    y = _bn_relu(y, s, ss, g1, bb1)
    y = _upsample(y)
    y, s, ss = _pconv(_pad_same4(y), Wd2, bd2, stats=True)
    y = _bn_relu(y, s, ss, g2, bb2)
    y = _upsample(y)
    y, s, ss = _pconv(_pad_same4(y), Wd3, bd3, stats=True)
    y = _bn_relu(y, s, ss, g3, bb3)

    rec = _pconv(y, Wo, bo, relu=True)
    return (rec, encoded, discrete, quantized)
